# baseline (speedup 1.0000x reference)
import functools

import jax
import jax.numpy as jnp
from jax import lax
from jax.experimental import pallas as pl
from jax.experimental.pallas import tpu as pltpu
from jax.experimental.pallas import tpu_sc as plsc

N_ATOMS = 10000
N_BONDS = 160000
MAX_NB = 16
ATOM_FDIM = 128
BOND_FDIM = 144
H = 256
DEPTH = 4
N_MOLS = 100
APM = 100

NC, NS = 2, 16
NW = NC * NS
E_PAD = 163840
N_PAD = 10240
BPW = E_PAD // NW
APW = N_PAD // NW
CB = 128
CA = 8
LC = H // 16

_MESH = dict(core_axis_name="c", subcore_axis_name="s")


def _mm_in_body(x_ref, wi_ref, wh_ref, inp_ref, m2_ref):
    inp = jnp.dot(x_ref[...], wi_ref[...], preferred_element_type=jnp.float32)
    inp_ref[...] = inp
    m = jnp.maximum(inp, 0.0)
    m2_ref[...] = jnp.dot(m, wh_ref[...], preferred_element_type=jnp.float32)


def _mm_in(fb, wi, wh):
    RB = 2048
    return pl.pallas_call(
        _mm_in_body,
        grid=(E_PAD // RB,),
        in_specs=[pl.BlockSpec((RB, BOND_FDIM), lambda i: (i, 0)),
                  pl.BlockSpec((BOND_FDIM, H), lambda i: (0, 0)),
                  pl.BlockSpec((H, H), lambda i: (0, 0))],
        out_specs=[pl.BlockSpec((RB, H), lambda i: (i, 0)),
                   pl.BlockSpec((RB, H), lambda i: (i, 0))],
        out_shape=[jax.ShapeDtypeStruct((E_PAD, H), jnp.float32),
                   jax.ShapeDtypeStruct((E_PAD, H), jnp.float32)],
    )(fb, wi, wh)


def _mm_h_body(inp_ref, g_ref, wh_ref, m2_ref):
    m = jnp.maximum(inp_ref[...] + g_ref[...], 0.0)
    m2_ref[...] = jnp.dot(m, wh_ref[...], preferred_element_type=jnp.float32)


def _mm_h(inp, g, wh):
    RB = 2048
    return pl.pallas_call(
        _mm_h_body,
        grid=(E_PAD // RB,),
        in_specs=[pl.BlockSpec((RB, H), lambda i: (i, 0)),
                  pl.BlockSpec((RB, H), lambda i: (i, 0)),
                  pl.BlockSpec((H, H), lambda i: (0, 0))],
        out_specs=pl.BlockSpec((RB, H), lambda i: (i, 0)),
        out_shape=jax.ShapeDtypeStruct((E_PAD, H), jnp.float32),
    )(inp, g, wh)


def _relu_add_body(inp_ref, g_ref, out_ref):
    out_ref[...] = jnp.maximum(inp_ref[...] + g_ref[...], 0.0)


def _relu_add(inp, g):
    RB = 4096
    return pl.pallas_call(
        _relu_add_body,
        grid=(E_PAD // RB,),
        in_specs=[pl.BlockSpec((RB, H), lambda i: (i, 0)),
                  pl.BlockSpec((RB, H), lambda i: (i, 0))],
        out_specs=pl.BlockSpec((RB, H), lambda i: (i, 0)),
        out_shape=jax.ShapeDtypeStruct((E_PAD, H), jnp.float32),
    )(inp, g)


def _final_body(fa_ref, am_ref, wo_ref, bo_ref, out_ref):
    wo = wo_ref[...]
    h = jnp.dot(fa_ref[...], wo[:ATOM_FDIM], preferred_element_type=jnp.float32)
    h = h + jnp.dot(am_ref[...], wo[ATOM_FDIM:], preferred_element_type=jnp.float32)
    h = jnp.maximum(h + bo_ref[...], 0.0)
    r = lax.broadcasted_iota(jnp.int32, (N_MOLS, N_PAD), 1) // APM
    m = lax.broadcasted_iota(jnp.int32, (N_MOLS, N_PAD), 0)
    sel = jnp.where(r == m, 1.0 / APM, 0.0)
    out_ref[...] = jnp.dot(sel, h, preferred_element_type=jnp.float32)


def _final(fa, am, wo, bo2):
    return pl.pallas_call(
        _final_body,
        out_shape=jax.ShapeDtypeStruct((N_MOLS, H), jnp.float32),
    )(fa, am, wo, bo2)


@functools.partial(
    pl.kernel,
    mesh=plsc.VectorSubcoreMesh(**_MESH),
    out_type=jax.ShapeDtypeStruct((N_PAD, H), jnp.float32),
    scratch_types=[
        pltpu.VMEM((CA * MAX_NB,), jnp.int32),
        pltpu.VMEM((CA * MAX_NB, H), jnp.float32),
        pltpu.VMEM((CA, H), jnp.float32),
        pltpu.SemaphoreType.DMA,
    ],
)
def _segsum(a2b_hbm, m2_hbm, out_hbm, idx_v, rows_v, acc_v, sem):
    wid = lax.axis_index("s") * NC + lax.axis_index("c")
    base = wid * APW

    def chunk(ci, _):
        a0 = base + ci * CA
        pltpu.sync_copy(a2b_hbm.at[pl.ds(a0 * MAX_NB, CA * MAX_NB)], idx_v)
        pltpu.async_copy(m2_hbm.at[idx_v], rows_v, sem).wait()

        def atom(a, _):
            r0 = a * MAX_NB
            for jj in range(LC):
                j = jj * 16
                acc = rows_v[r0, pl.ds(j, 16)]
                for k in range(1, MAX_NB):
                    acc = acc + rows_v[r0 + k, pl.ds(j, 16)]
                acc_v[a, pl.ds(j, 16)] = acc
            return 0

        lax.fori_loop(0, CA, atom, 0)
        pltpu.sync_copy(acc_v, out_hbm.at[pl.ds(a0, CA)])
        return 0

    lax.fori_loop(0, APW // CA, chunk, 0)


@functools.partial(
    pl.kernel,
    mesh=plsc.VectorSubcoreMesh(**_MESH),
    out_type=jax.ShapeDtypeStruct((E_PAD, H), jnp.float32),
    scratch_types=[
        pltpu.VMEM((CB,), jnp.int32),
        pltpu.VMEM((CB,), jnp.int32),
        pltpu.VMEM((CB, H), jnp.float32),
        pltpu.VMEM((CB, H), jnp.float32),
        pltpu.SemaphoreType.DMA,
        pltpu.SemaphoreType.DMA,
    ],
)
def _gather_sub(b2a_hbm, b2revb_hbm, amsg_hbm, m2_hbm, out_hbm,
                idxa_v, idxr_v, ga_v, gr_v, sema, semr):
    wid = lax.axis_index("s") * NC + lax.axis_index("c")
    base = wid * BPW

    def chunk(ci, _):
        e0 = base + ci * CB
        pltpu.sync_copy(b2a_hbm.at[pl.ds(e0, CB)], idxa_v)
        pltpu.sync_copy(b2revb_hbm.at[pl.ds(e0, CB)], idxr_v)
        cpa = pltpu.async_copy(amsg_hbm.at[idxa_v], ga_v, sema)
        cpr = pltpu.async_copy(m2_hbm.at[idxr_v], gr_v, semr)
        cpa.wait()
        cpr.wait()

        def row(r, _):
            for jj in range(LC):
                j = jj * 16
                ga_v[r, pl.ds(j, 16)] = (ga_v[r, pl.ds(j, 16)]
                                         - gr_v[r, pl.ds(j, 16)])
            return 0

        lax.fori_loop(0, CB, row, 0)
        pltpu.sync_copy(ga_v, out_hbm.at[pl.ds(e0, CB)])
        return 0

    lax.fori_loop(0, BPW // CB, chunk, 0)


def kernel(f_atoms, f_bonds, a2b, b2a, b2revb, W_i, W_h, W_o, b_o):
    a2b_flat = jnp.pad(a2b.astype(jnp.int32).reshape(-1),
                       (0, (N_PAD - N_ATOMS) * MAX_NB))
    b2a_p = jnp.pad(b2a.astype(jnp.int32), (0, E_PAD - N_BONDS))
    b2revb_p = jnp.pad(b2revb.astype(jnp.int32), (0, E_PAD - N_BONDS))
    fb_p = jnp.pad(f_bonds, ((0, E_PAD - N_BONDS), (0, 0)))
    fa_p = jnp.pad(f_atoms, ((0, N_PAD - N_ATOMS), (0, 0)))

    inp, m2 = _mm_in(fb_p, W_i, W_h)
    g = None
    for it in range(DEPTH - 1):
        amsg2 = _segsum(a2b_flat, m2)
        g = _gather_sub(b2a_p, b2revb_p, amsg2, m2)
        if it < DEPTH - 2:
            m2 = _mm_h(inp, g, W_h)
    msg = _relu_add(inp, g)
    amsg = _segsum(a2b_flat, msg)
    return _final(fa_p, amsg, W_o, b_o.reshape(1, H))
